# Initial kernel scaffold; baseline (speedup 1.0000x reference)
import dataclasses
import functools

import jax
import jax.numpy as jnp
from jax import lax
from jax.experimental import pallas as pl
from jax.experimental.pallas import tpu as pltpu
from jax.experimental.pallas import tpu_sc as plsc

N = 10000
E = 320000
D = 128
NL = 7

NC = 2
NS = 16
NW = NC * NS
NP = 10240
R = NP // NW
PC = 1600
C2 = 128
EL = E + (E // PC) * 8 + 4 * C2

_NEG_INF = jnp.float32(float("-inf"))


def _compiler_params():
    cp = pltpu.CompilerParams()
    if "needs_layout_passes" in pltpu.CompilerParams.__dataclass_fields__:
        cp = dataclasses.replace(cp, needs_layout_passes=False)
    return cp


def _worker_id():
    return lax.axis_index("s") * NC + lax.axis_index("c")


def _make_partition():
    mesh = plsc.VectorSubcoreMesh(core_axis_name="c", subcore_axis_name="s")

    @functools.partial(
        pl.kernel,
        mesh=mesh,
        out_type=[
            jax.ShapeDtypeStruct((NW, EL), jnp.int32),
            jax.ShapeDtypeStruct((NW, EL), jnp.int32),
            jax.ShapeDtypeStruct((NW, 8), jnp.int32),
        ],
        scratch_types=[
            pltpu.VMEM((PC,), jnp.int32),
            pltpu.VMEM((PC,), jnp.int32),
            pltpu.VMEM((PC + 16,), jnp.int32),
            pltpu.VMEM((PC + 16,), jnp.int32),
            pltpu.SMEM((8,), jnp.int32),
            pltpu.SemaphoreType.DMA,
        ],
        compiler_params=_compiler_params(),
    )
    def partition(ei_hbm, lsrc_hbm, ldst_hbm, cnt_hbm,
                  src_v, dst_v, csrc_v, cldst_v, cnt_s, sem):
        wid = _worker_id()
        lo = wid * R
        hi = lo + R
        iota = lax.iota(jnp.int32, 16)

        def chunk_body(c, off):
            base = c * PC
            pltpu.async_copy(ei_hbm.at[0, pl.ds(base, PC)], src_v, sem).wait()
            pltpu.async_copy(ei_hbm.at[1, pl.ds(base, PC)], dst_v, sem).wait()

            def vec_body(v, local_off):
                d = dst_v[pl.ds(v * 16, 16)]
                s = src_v[pl.ds(v * 16, 16)]
                m = jnp.logical_and(d >= lo, d < hi)
                mi = m.astype(jnp.int32)
                pc = plsc.cumsum(mi)
                pos = local_off + pc - 1
                plsc.store_scatter(csrc_v, [pos], s, mask=m)
                plsc.store_scatter(cldst_v, [pos], d - lo, mask=m)
                return local_off + jnp.sum(mi)

            local = lax.fori_loop(0, PC // 16, vec_body, jnp.int32(0))
            aligned = jnp.bitwise_and(local + 7, jnp.int32(-8))
            padm = iota < (aligned - local)
            plsc.store_scatter(csrc_v, [local + iota],
                               jnp.full((16,), lo, jnp.int32), mask=padm)
            plsc.store_scatter(cldst_v, [local + iota],
                               jnp.full((16,), R, jnp.int32), mask=padm)
            pltpu.async_copy(csrc_v.at[pl.ds(0, PC)],
                             lsrc_hbm.at[wid, pl.ds(off, PC)], sem).wait()
            pltpu.async_copy(cldst_v.at[pl.ds(0, PC)],
                             ldst_hbm.at[wid, pl.ds(off, PC)], sem).wait()
            return off + aligned

        off = lax.fori_loop(0, E // PC, chunk_body, jnp.int32(0))

        for j in range(C2 // 16):
            csrc_v[pl.ds(j * 16, 16)] = jnp.full((16,), lo, jnp.int32)
            cldst_v[pl.ds(j * 16, 16)] = jnp.full((16,), R, jnp.int32)
        pltpu.async_copy(csrc_v.at[pl.ds(0, C2)],
                         lsrc_hbm.at[wid, pl.ds(off, C2)], sem).wait()
        pltpu.async_copy(cldst_v.at[pl.ds(0, C2)],
                         ldst_hbm.at[wid, pl.ds(off, C2)], sem).wait()
        count = jnp.bitwise_and(off + (C2 - 1), jnp.int32(-C2))
        for j in range(8):
            cnt_s[j] = count
        pltpu.async_copy(cnt_s, cnt_hbm.at[wid], sem).wait()

    return partition


def _make_segmax():
    mesh = plsc.VectorSubcoreMesh(core_axis_name="c", subcore_axis_name="s")

    @functools.partial(
        pl.kernel,
        mesh=mesh,
        out_type=jax.ShapeDtypeStruct((NP, D), jnp.float32),
        scratch_types=[
            pltpu.VMEM((C2,), jnp.int32),
            pltpu.VMEM((C2, D), jnp.float32),
            pltpu.VMEM((R + 1, D), jnp.float32),
            pltpu.SMEM((C2,), jnp.int32),
            pltpu.SMEM((8,), jnp.int32),
            pltpu.SemaphoreType.DMA,
            pltpu.SemaphoreType.DMA,
        ],
        compiler_params=_compiler_params(),
    )
    def segmax(h_hbm, lsrc_hbm, ldst_hbm, cnt_hbm, out_hbm,
               idx_v, rows_v, agg_v, ldst_s, cnt_s, sem, gsem):
        wid = _worker_id()
        base = wid * R

        @pl.loop(0, R + 1)
        def _(r):
            for k in range(D // 16):
                agg_v[r, pl.ds(k * 16, 16)] = jnp.full((16,), _NEG_INF)

        pltpu.async_copy(cnt_hbm.at[wid], cnt_s, sem).wait()
        nchunks = cnt_s[0] // C2

        def chunk_body(c, carry):
            eb = c * C2
            pltpu.async_copy(lsrc_hbm.at[wid, pl.ds(eb, C2)], idx_v, sem).wait()
            pltpu.async_copy(ldst_hbm.at[wid, pl.ds(eb, C2)], ldst_s, sem).wait()
            pltpu.async_copy(h_hbm.at[idx_v], rows_v, gsem).wait()

            @pl.loop(0, C2)
            def _(e):
                d = ldst_s[e]
                for k in range(D // 16):
                    sl = pl.ds(k * 16, 16)
                    agg_v[d, sl] = jnp.maximum(agg_v[d, sl], rows_v[e, sl])

            return carry

        lax.fori_loop(0, nchunks, chunk_body, jnp.int32(0))

        @pl.loop(0, R)
        def _(r):
            for k in range(D // 16):
                sl = pl.ds(k * 16, 16)
                v = agg_v[r, sl]
                agg_v[r, sl] = jnp.where(v == _NEG_INF, jnp.float32(0.0), v)

        pltpu.async_copy(agg_v.at[pl.ds(0, R)],
                         out_hbm.at[pl.ds(base, R)], sem).wait()

    return segmax


_partition_fn = _make_partition()
_segmax_fn = _make_segmax()


_BR = 1024


def _tc_body_leaky(agg_ref, h_ref, wl_ref, bl_ref, wr_ref, o_ref):
    acc = lax.dot_general(agg_ref[...], wl_ref[...],
                          (((1,), (1,)), ((), ())),
                          preferred_element_type=jnp.float32)
    acc = acc + lax.dot_general(h_ref[...], wr_ref[...],
                                (((1,), (1,)), ((), ())),
                                preferred_element_type=jnp.float32)
    acc = acc + bl_ref[...]
    o_ref[...] = jnp.where(acc >= 0, acc, jnp.float32(0.02) * acc)


def _tc_body_final(agg_ref, h_ref, wl_ref, bl_ref, wr_ref, o_ref):
    acc = lax.dot_general(agg_ref[...], wl_ref[...],
                          (((1,), (1,)), ((), ())),
                          preferred_element_type=jnp.float32)
    acc = acc + lax.dot_general(h_ref[...], wr_ref[...],
                                (((1,), (1,)), ((), ())),
                                preferred_element_type=jnp.float32)
    acc = acc + bl_ref[...]
    o_ref[...] = jnp.tanh(acc) * jnp.float32(0.5)


def _tc_layer(agg, h, Wl, bl, Wr, final):
    body = _tc_body_final if final else _tc_body_leaky
    return pl.pallas_call(
        body,
        out_shape=jax.ShapeDtypeStruct((NP, D), jnp.float32),
        grid=(NP // _BR,),
        in_specs=[
            pl.BlockSpec((_BR, D), lambda i: (i, 0)),
            pl.BlockSpec((_BR, D), lambda i: (i, 0)),
            pl.BlockSpec((D, D), lambda i: (0, 0)),
            pl.BlockSpec((1, D), lambda i: (0, 0)),
            pl.BlockSpec((D, D), lambda i: (0, 0)),
        ],
        out_specs=pl.BlockSpec((_BR, D), lambda i: (i, 0)),
    )(agg, h, Wl, bl, Wr)


def kernel(x, edge_index, Wl_sage, bl_sage, Wr_sage, Wl_out, bl_out, Wr_out):
    x_pad = jnp.zeros((NP, D), jnp.float32).at[:N].set(x)
    lsrc, ldst, counts = _partition_fn(edge_index)
    h = x_pad
    for i in range(NL):
        agg = _segmax_fn(h, lsrc, ldst, counts)
        h = _tc_layer(agg, h, Wl_sage[i], jnp.reshape(bl_sage[i], (1, D)),
                      Wr_sage[i], final=False)
    agg = _segmax_fn(h, lsrc, ldst, counts)
    h = _tc_layer(agg, h, Wl_out, jnp.reshape(bl_out, (1, D)), Wr_out,
                  final=True)
    return h[:N]


# speedup vs baseline: 2.1715x; 2.1715x over previous
import dataclasses
import functools

import jax
import jax.numpy as jnp
from jax import lax
from jax.experimental import pallas as pl
from jax.experimental.pallas import tpu as pltpu
from jax.experimental.pallas import tpu_sc as plsc

N = 10000
E = 320000
D = 128
NL = 7

NC = 2
NS = 16
NW = NC * NS
NP = 10240
R = NP // NW
PC = 1600
C2 = 128
EL = E + (E // PC) * 8 + 4 * C2

_NEG_INF = float("-inf")


def _compiler_params():
    cp = pltpu.CompilerParams()
    if "needs_layout_passes" in pltpu.CompilerParams.__dataclass_fields__:
        cp = dataclasses.replace(cp, needs_layout_passes=False)
    return cp


def _worker_id():
    return lax.axis_index("s") * NC + lax.axis_index("c")


def _make_partition():
    mesh = plsc.VectorSubcoreMesh(core_axis_name="c", subcore_axis_name="s")

    @functools.partial(
        pl.kernel,
        mesh=mesh,
        out_type=[
            jax.ShapeDtypeStruct((NW * EL,), jnp.int32),
            jax.ShapeDtypeStruct((NW * EL,), jnp.int32),
            jax.ShapeDtypeStruct((NW * 8,), jnp.int32),
        ],
        scratch_types=[
            pltpu.VMEM((PC,), jnp.int32),
            pltpu.VMEM((PC,), jnp.int32),
            pltpu.VMEM((PC + 16,), jnp.int32),
            pltpu.VMEM((PC + 16,), jnp.int32),
            pltpu.VMEM((16,), jnp.int32),
            pltpu.SemaphoreType.DMA,
        ],
        compiler_params=_compiler_params(),
    )
    def partition(src_hbm, dst_hbm, lsrc_hbm, ldst_hbm, cnt_hbm,
                  src_v, dst_v, csrc_v, cldst_v, cnt_v, sem):
        wid = _worker_id()
        lo = wid * R
        hi = lo + R
        lbase = wid * EL
        iota = lax.iota(jnp.int32, 16)

        def chunk_body(c, off):
            base = c * PC
            pltpu.async_copy(src_hbm.at[pl.ds(base, PC)], src_v, sem).wait()
            pltpu.async_copy(dst_hbm.at[pl.ds(base, PC)], dst_v, sem).wait()

            def vec_body(v, local_off):
                d = dst_v[pl.ds(v * 16, 16)]
                s = src_v[pl.ds(v * 16, 16)]
                m = jnp.logical_and(d >= lo, d < hi)
                mi = m.astype(jnp.int32)
                pc = plsc.cumsum(mi)
                pos = local_off + pc - 1
                plsc.store_scatter(csrc_v, [pos], s, mask=m)
                plsc.store_scatter(cldst_v, [pos], d - lo, mask=m)
                return local_off + jnp.sum(mi)

            local = lax.fori_loop(0, PC // 16, vec_body, jnp.int32(0))
            aligned = jnp.bitwise_and(local + 7, jnp.int32(-8))
            padm = iota < (aligned - local)
            plsc.store_scatter(csrc_v, [local + iota],
                               jnp.full((16,), lo, jnp.int32), mask=padm)
            plsc.store_scatter(cldst_v, [local + iota],
                               jnp.full((16,), R, jnp.int32), mask=padm)
            pltpu.async_copy(csrc_v.at[pl.ds(0, PC)],
                             lsrc_hbm.at[pl.ds(pl.multiple_of(lbase + off, 8), PC)], sem).wait()
            pltpu.async_copy(cldst_v.at[pl.ds(0, PC)],
                             ldst_hbm.at[pl.ds(pl.multiple_of(lbase + off, 8), PC)], sem).wait()
            return off + aligned

        off = lax.fori_loop(0, E // PC, chunk_body, jnp.int32(0))

        for j in range(C2 // 16):
            csrc_v[pl.ds(j * 16, 16)] = jnp.full((16,), lo, jnp.int32)
            cldst_v[pl.ds(j * 16, 16)] = jnp.full((16,), R, jnp.int32)
        pltpu.async_copy(csrc_v.at[pl.ds(0, C2)],
                         lsrc_hbm.at[pl.ds(pl.multiple_of(lbase + off, 8), C2)], sem).wait()
        pltpu.async_copy(cldst_v.at[pl.ds(0, C2)],
                         ldst_hbm.at[pl.ds(pl.multiple_of(lbase + off, 8), C2)], sem).wait()
        count = jnp.bitwise_and(off + (C2 - 1), jnp.int32(-C2))
        cnt_v[pl.ds(0, 16)] = jnp.full((16,), count, jnp.int32)
        pltpu.async_copy(cnt_v.at[pl.ds(0, 8)],
                         cnt_hbm.at[pl.ds(pl.multiple_of(wid * 8, 8), 8)],
                         sem).wait()

    return partition


def _make_segmax():
    mesh = plsc.VectorSubcoreMesh(core_axis_name="c", subcore_axis_name="s")

    @functools.partial(
        pl.kernel,
        mesh=mesh,
        out_type=jax.ShapeDtypeStruct((NP, D), jnp.float32),
        scratch_types=[
            pltpu.VMEM((C2,), jnp.int32),
            pltpu.VMEM((C2, D), jnp.float32),
            pltpu.VMEM((R + 1, D), jnp.float32),
            pltpu.VMEM((C2,), jnp.int32),
            pltpu.VMEM((16,), jnp.int32),
            pltpu.SemaphoreType.DMA,
            pltpu.SemaphoreType.DMA,
        ],
        compiler_params=_compiler_params(),
    )
    def segmax(h_hbm, lsrc_hbm, ldst_hbm, cnt_hbm, out_hbm,
               idx_v, rows_v, agg_v, ldst_v, cnt_v, sem, gsem):
        wid = _worker_id()
        base = wid * R
        lbase = wid * EL

        @pl.loop(0, R + 1)
        def _(r):
            for k in range(D // 16):
                agg_v[r, pl.ds(k * 16, 16)] = jnp.full((16,), _NEG_INF,
                                                       jnp.float32)

        pltpu.async_copy(cnt_hbm.at[pl.ds(pl.multiple_of(wid * 8, 8), 8)],
                         cnt_v.at[pl.ds(0, 8)], sem).wait()
        nchunks = cnt_v[pl.ds(0, 16)][0] // C2

        def chunk_body(c, carry):
            eb = lbase + c * C2
            pltpu.async_copy(lsrc_hbm.at[pl.ds(pl.multiple_of(eb, 8), C2)], idx_v, sem).wait()
            pltpu.async_copy(ldst_hbm.at[pl.ds(pl.multiple_of(eb, 8), C2)], ldst_v, sem).wait()
            pltpu.async_copy(h_hbm.at[idx_v], rows_v, gsem).wait()

            @pl.loop(0, C2 // 16)
            def _(vi):
                dvec = ldst_v[pl.ds(vi * 16, 16)]
                for l in range(16):
                    d = dvec[l]
                    e = vi * 16 + l
                    for k in range(D // 16):
                        sl = pl.ds(k * 16, 16)
                        agg_v[d, sl] = jnp.maximum(agg_v[d, sl],
                                                   rows_v[e, sl])

            return carry

        lax.fori_loop(0, nchunks, chunk_body, jnp.int32(0))

        @pl.loop(0, R)
        def _(r):
            for k in range(D // 16):
                sl = pl.ds(k * 16, 16)
                v = agg_v[r, sl]
                agg_v[r, sl] = jnp.where(v == _NEG_INF, jnp.float32(0.0), v)

        pltpu.async_copy(agg_v.at[pl.ds(0, R)],
                         out_hbm.at[pl.ds(base, R)], sem).wait()

    return segmax


_partition_fn = _make_partition()
_segmax_fn = _make_segmax()


_BR = 1024


def _tc_body_leaky(agg_ref, h_ref, wl_ref, bl_ref, wr_ref, o_ref):
    acc = lax.dot_general(agg_ref[...], wl_ref[...],
                          (((1,), (1,)), ((), ())),
                          preferred_element_type=jnp.float32)
    acc = acc + lax.dot_general(h_ref[...], wr_ref[...],
                                (((1,), (1,)), ((), ())),
                                preferred_element_type=jnp.float32)
    acc = acc + bl_ref[...]
    o_ref[...] = jnp.where(acc >= 0, acc, jnp.float32(0.02) * acc)


def _tc_body_final(agg_ref, h_ref, wl_ref, bl_ref, wr_ref, o_ref):
    acc = lax.dot_general(agg_ref[...], wl_ref[...],
                          (((1,), (1,)), ((), ())),
                          preferred_element_type=jnp.float32)
    acc = acc + lax.dot_general(h_ref[...], wr_ref[...],
                                (((1,), (1,)), ((), ())),
                                preferred_element_type=jnp.float32)
    acc = acc + bl_ref[...]
    o_ref[...] = jnp.tanh(acc) * jnp.float32(0.5)


def _tc_layer(agg, h, Wl, bl, Wr, final):
    body = _tc_body_final if final else _tc_body_leaky
    return pl.pallas_call(
        body,
        out_shape=jax.ShapeDtypeStruct((NP, D), jnp.float32),
        grid=(NP // _BR,),
        in_specs=[
            pl.BlockSpec((_BR, D), lambda i: (i, 0)),
            pl.BlockSpec((_BR, D), lambda i: (i, 0)),
            pl.BlockSpec((D, D), lambda i: (0, 0)),
            pl.BlockSpec((1, D), lambda i: (0, 0)),
            pl.BlockSpec((D, D), lambda i: (0, 0)),
        ],
        out_specs=pl.BlockSpec((_BR, D), lambda i: (i, 0)),
    )(agg, h, Wl, bl, Wr)


def kernel(x, edge_index, Wl_sage, bl_sage, Wr_sage, Wl_out, bl_out, Wr_out):
    x_pad = jnp.zeros((NP, D), jnp.float32).at[:N].set(x)
    lsrc, ldst, counts = _partition_fn(edge_index[0], edge_index[1])
    h = x_pad
    for i in range(NL):
        agg = _segmax_fn(h, lsrc, ldst, counts)
        h = _tc_layer(agg, h, Wl_sage[i], jnp.reshape(bl_sage[i], (1, D)),
                      Wr_sage[i], final=False)
    agg = _segmax_fn(h, lsrc, ldst, counts)
    h = _tc_layer(agg, h, Wl_out, jnp.reshape(bl_out, (1, D)), Wr_out,
                  final=True)
    return h[:N]
